# baseline (speedup 1.0000x reference)
import jax
import jax.numpy as jnp
from jax import lax
from jax.experimental import pallas as pl
from jax.experimental.pallas import tpu as pltpu
from jax.experimental.pallas import tpu_sc as plsc

VOCAB = 100000
D = 128
SEQ = 200
BATCH = 1024
TOK = BATCH * SEQ
NC, NS, L = 2, 16, 16
NW = NC * NS
PER_W = TOK // NW
R = 100
MR = SEQ
SPM = MR // R
MCHUNKS = PER_W // MR
NBUF = 3


def _pos_encoding():
    pos = jnp.arange(0, SEQ, dtype=jnp.float32)[:, None]
    _2i = jnp.arange(0, D, 2, dtype=jnp.float32)
    angles = pos / jnp.power(10000.0, _2i / D)
    enc = jnp.zeros((SEQ, D), dtype=jnp.float32)
    enc = enc.at[:, 0::2].set(jnp.sin(angles))
    enc = enc.at[:, 1::2].set(jnp.cos(angles))
    return enc


def _body(table_hbm, idx_hbm, pos_hbm, out_hbm, idx_v, pos_v, rows, gsems, wsems):
    wid = lax.axis_index("s") * NC + lax.axis_index("c")
    base = wid * MCHUNKS * SPM
    head = (NBUF - 1) * SPM
    pltpu.sync_copy(idx_hbm.at[pl.ds(base, head)], idx_v.at[pl.ds(0, head)])
    rest_d = pltpu.async_copy(
        idx_hbm.at[pl.ds(base + head, MCHUNKS * SPM - head)],
        idx_v.at[pl.ds(head, MCHUNKS * SPM - head)],
        wsems[0],
    )
    pos_d = pltpu.async_copy(pos_hbm, pos_v, wsems[1])

    def start_gather(g):
        b = g % NBUF
        return [
            pltpu.async_copy(
                table_hbm.at[idx_v.at[g * SPM + s]],
                rows[b].at[pl.ds(s * R, R)],
                gsems[b],
            )
            for s in range(SPM)
        ]

    gather_d = [None] * MCHUNKS
    write_d = [None] * MCHUNKS
    for g in range(NBUF - 1):
        gather_d[g] = start_gather(g)
    rest_d.wait()
    pos_d.wait()

    for c in range(MCHUNKS):
        b = c % NBUF
        for d_ in gather_d[c]:
            d_.wait()

        @plsc.parallel_loop(0, MR)
        def add_body(i, b=b):
            for j in range(D // L):
                sl = pl.ds(j * L, L)
                plsc.addupdate(rows[b].at[i, sl], pos_v[i, sl])

        write_d[c] = pltpu.async_copy(
            rows[b], out_hbm.at[pl.ds((wid * MCHUNKS + c) * MR, MR)], wsems[b]
        )
        g = c + NBUF - 1
        if g < MCHUNKS:
            if g >= NBUF:
                write_d[g - NBUF].wait()
            gather_d[g] = start_gather(g)
    for c in range(MCHUNKS - NBUF, MCHUNKS):
        write_d[c].wait()


@jax.jit
def kernel(x, table):
    pos = _pos_encoding()
    idx = x.reshape(TOK // R, R)
    mesh = plsc.VectorSubcoreMesh(core_axis_name="c", subcore_axis_name="s")
    out = pl.kernel(
        _body,
        out_type=jax.ShapeDtypeStruct((TOK, D), jnp.float32),
        mesh=mesh,
        scratch_types=[
            pltpu.VMEM((MCHUNKS * SPM, R), jnp.int32),
            pltpu.VMEM((SEQ, D), jnp.float32),
            [pltpu.VMEM((MR, D), jnp.float32) for _ in range(NBUF)],
            [pltpu.SemaphoreType.DMA for _ in range(NBUF)],
            [pltpu.SemaphoreType.DMA for _ in range(NBUF)],
        ],
        compiler_params=pltpu.CompilerParams(
            use_tc_tiling_on_sc=False,
            disable_bounds_checks=True,
            disable_semaphore_checks=True,
        ),
    )(table, idx, pos)
    return out.reshape(BATCH, SEQ, D)


# speedup vs baseline: 1.0089x; 1.0089x over previous
import jax
import jax.numpy as jnp
from jax import lax
from jax.experimental import pallas as pl
from jax.experimental.pallas import tpu as pltpu
from jax.experimental.pallas import tpu_sc as plsc

VOCAB = 100000
D = 128
SEQ = 200
BATCH = 1024
TOK = BATCH * SEQ
NC, NS, L = 2, 16, 16
NW = NC * NS
PER_W = TOK // NW
R = 100
CHUNKS = PER_W // R
NBUF = 4


def _pos_encoding():
    pos = jnp.arange(0, SEQ, dtype=jnp.float32)[:, None]
    _2i = jnp.arange(0, D, 2, dtype=jnp.float32)
    angles = pos / jnp.power(10000.0, _2i / D)
    enc = jnp.zeros((SEQ, D), dtype=jnp.float32)
    enc = enc.at[:, 0::2].set(jnp.sin(angles))
    enc = enc.at[:, 1::2].set(jnp.cos(angles))
    return enc


def _body(table_hbm, idx_hbm, pos_hbm, out_hbm, idx_v, pos_v, rows, gsems, wsems):
    wid = lax.axis_index("s") * NC + lax.axis_index("c")
    base = wid * CHUNKS
    head = NBUF - 1
    pltpu.sync_copy(idx_hbm.at[pl.ds(base, head)], idx_v.at[pl.ds(0, head)])
    rest_d = pltpu.async_copy(
        idx_hbm.at[pl.ds(base + head, CHUNKS - head)],
        idx_v.at[pl.ds(head, CHUNKS - head)],
        wsems[0],
    )
    pos_d = pltpu.async_copy(pos_hbm, pos_v, wsems[1])

    def start_gather(g):
        b = g % NBUF
        return pltpu.async_copy(table_hbm.at[idx_v.at[g]], rows[b], gsems[b])

    gather_d = [None] * CHUNKS
    write_d = [None] * CHUNKS
    for g in range(NBUF - 1):
        gather_d[g] = start_gather(g)
    rest_d.wait()
    pos_d.wait()

    for c in range(CHUNKS):
        b = c % NBUF
        gather_d[c].wait()
        ph = (c % 2) * R

        @plsc.parallel_loop(0, R)
        def add_body(i, ph=ph, b=b):
            for j in range(D // L):
                sl = pl.ds(j * L, L)
                plsc.addupdate(rows[b].at[i, sl], pos_v[ph + i, sl])

        write_d[c] = pltpu.async_copy(
            rows[b], out_hbm.at[pl.ds((base + c) * R, R)], wsems[b]
        )
        g = c + NBUF - 1
        if g < CHUNKS:
            if g >= NBUF:
                write_d[g - NBUF].wait()
            gather_d[g] = start_gather(g)
    for c in range(CHUNKS - NBUF, CHUNKS):
        write_d[c].wait()


@jax.jit
def kernel(x, table):
    pos = _pos_encoding()
    idx = x.reshape(TOK // R, R)
    mesh = plsc.VectorSubcoreMesh(core_axis_name="c", subcore_axis_name="s")
    out = pl.kernel(
        _body,
        out_type=jax.ShapeDtypeStruct((TOK, D), jnp.float32),
        mesh=mesh,
        scratch_types=[
            pltpu.VMEM((CHUNKS, R), jnp.int32),
            pltpu.VMEM((SEQ, D), jnp.float32),
            [pltpu.VMEM((R, D), jnp.float32) for _ in range(NBUF)],
            [pltpu.SemaphoreType.DMA for _ in range(NBUF)],
            [pltpu.SemaphoreType.DMA for _ in range(NBUF)],
        ],
        compiler_params=pltpu.CompilerParams(use_tc_tiling_on_sc=False),
    )(table, idx, pos)
    return out.reshape(BATCH, SEQ, D)
